# baseline (speedup 1.0000x reference)
import functools

import jax
import jax.numpy as jnp
from jax import lax
from jax.experimental import pallas as pl
from jax.experimental.pallas import tpu as pltpu
from jax.experimental.pallas import tpu_sc as plsc

N = 10000
E = 320000
D = 128

NC = 2
NS = 16
NW = NC * NS
EPW = E // NW
RPTA = 640
RPTB = N - 15 * RPTA


def _per_tile_rows(sid, fn):
    @pl.when(sid < NS - 1)
    def _():
        fn(sid * RPTA, RPTA)

    @pl.when(sid == NS - 1)
    def _():
        fn((NS - 1) * RPTA, RPTB)


_mesh = plsc.VectorSubcoreMesh(core_axis_name="c", subcore_axis_name="s")


K = 80
NCHUNK = EPW // K


def _fill_rows(buf, value):
    def body(j, _):
        for c in range(D // 16):
            buf[j, pl.ds(c * 16, 16)] = jnp.full((16,), value, jnp.float32)
        return 0
    lax.fori_loop(0, K, body, 0)


def _zero_table(sid, zbuf, table_sh):
    def zero_rows(r0, nr):
        for j in range(nr // K):
            pltpu.sync_copy(zbuf, table_sh.at[pl.ds(r0 + j * K, K)])
    _per_tile_rows(sid, zero_rows)


def _copy_out(sid, cid, table_sh, vbuf, out_h):
    def out_rows(r0, nr):
        for j in range(nr // K):
            pltpu.sync_copy(table_sh.at[pl.ds(r0 + j * K, K)], vbuf)
            pltpu.sync_copy(vbuf, out_h.at[cid, pl.ds(r0 + j * K, K)])
    _per_tile_rows(sid, out_rows)


@functools.partial(
    pl.kernel,
    out_type=[jax.ShapeDtypeStruct((NC, N, D), jnp.float32)],
    mesh=_mesh,
    scratch_types=[
        pltpu.VMEM((K,), jnp.int32),
        pltpu.VMEM((K,), jnp.int32),
        pltpu.VMEM((K, D), jnp.float32),
        pltpu.VMEM((K, 2 * D), jnp.float32),
        pltpu.VMEM_SHARED((N, D), jnp.float32),
        pltpu.SemaphoreType.DMA,
        pltpu.SemaphoreType.DMA,
    ],
)
def _edge_kernel(src_h, dst_h, xw_h, f_h, p_h,
                 sidx, didx, xj, fv, agg_sh, sem1, sem2):
    cid = lax.axis_index("c")
    sid = lax.axis_index("s")
    wid = sid * NC + cid

    _fill_rows(xj, 0.0)
    _zero_table(sid, xj, agg_sh)
    plsc.subcore_barrier()

    base = wid * EPW

    def chunk_body(i, carry):
        off = base + i * K
        pltpu.sync_copy(src_h.at[pl.ds(off, K)], sidx)
        pltpu.sync_copy(dst_h.at[pl.ds(off, K)], didx)
        cp1 = pltpu.async_copy(xw_h.at[sidx], xj, sem1)
        cp2 = pltpu.async_copy(f_h.at[didx], fv, sem2)
        cp1.wait()
        cp2.wait()

        @plsc.parallel_loop(0, K, 1, unroll=4)
        def _(e):
            for c in range(D // 16):
                s = c * 16
                xjv = xj[e, pl.ds(s, 16)]
                b = fv[e, pl.ds(s, 16)]
                g = fv[e, pl.ds(D + s, 16)]
                xj[e, pl.ds(s, 16)] = jnp.maximum(g * xjv + b, 0.0)

        pltpu.sync_copy(xj, agg_sh.at[didx], add=True)
        return carry

    lax.fori_loop(0, NCHUNK, chunk_body, 0)
    plsc.subcore_barrier()
    _copy_out(sid, cid, agg_sh, xj, p_h)


@functools.partial(
    pl.kernel,
    out_type=[jax.ShapeDtypeStruct((NC, N, D), jnp.float32)],
    mesh=_mesh,
    scratch_types=[
        pltpu.VMEM((K,), jnp.int32),
        pltpu.VMEM((K, D), jnp.float32),
        pltpu.VMEM_SHARED((N, D), jnp.float32),
    ],
)
def _cnt_kernel(dst_h, cnt_h, didx, buf, cnt_sh):
    cid = lax.axis_index("c")
    sid = lax.axis_index("s")
    wid = sid * NC + cid

    _fill_rows(buf, 0.0)
    _zero_table(sid, buf, cnt_sh)
    _fill_rows(buf, 1.0)
    plsc.subcore_barrier()

    base = wid * EPW

    def chunk_body(i, carry):
        pltpu.sync_copy(dst_h.at[pl.ds(base + i * K, K)], didx)
        pltpu.sync_copy(buf, cnt_sh.at[didx], add=True)
        return carry

    lax.fori_loop(0, NCHUNK, chunk_body, 0)
    plsc.subcore_barrier()
    _copy_out(sid, cid, cnt_sh, buf, cnt_h)


R = 1000


def _tc_common(h, wcat_ref, bf_ref, bfs_ref, xw_ref, f_ref, out_ref):
    r = jnp.dot(h, wcat_ref[...], preferred_element_type=jnp.float32)
    xw_ref[...] = r[:, 0:D]
    sk = r[:, D:2 * D]
    f_ref[...] = r[:, 2 * D:4 * D] + bf_ref[...]
    fs = r[:, 4 * D:6 * D] + bfs_ref[...]
    out_ref[...] = jnp.maximum(fs[:, D:2 * D] * sk + fs[:, 0:D], 0.0)


def _tc_first_body(x_ref, wcat_ref, bf_ref, bfs_ref, xw_ref, f_ref, out_ref):
    _tc_common(x_ref[...], wcat_ref, bf_ref, bfs_ref, xw_ref, f_ref, out_ref)


def _combine(outp_ref, p0_ref, p1_ref, c0_ref, c1_ref):
    cnt = c0_ref[0, :, 0:1] + c1_ref[0, :, 0:1]
    inv = 1.0 / jnp.maximum(cnt, 1.0)
    return outp_ref[...] + (p0_ref[0] + p1_ref[0]) * inv


def _tc_fuse_body(outp_ref, p0_ref, p1_ref, c0_ref, c1_ref,
                  wcat_ref, bf_ref, bfs_ref, xw_ref, f_ref, out_ref):
    h = jnp.maximum(_combine(outp_ref, p0_ref, p1_ref, c0_ref, c1_ref), 0.0)
    _tc_common(h, wcat_ref, bf_ref, bfs_ref, xw_ref, f_ref, out_ref)


def _tc_final_body(outp_ref, p0_ref, p1_ref, c0_ref, c1_ref, h_ref):
    h_ref[...] = _combine(outp_ref, p0_ref, p1_ref, c0_ref, c1_ref)


_w_spec = pl.BlockSpec((D, 6 * D), lambda i: (0, 0))
_b_spec = pl.BlockSpec((1, 2 * D), lambda i: (0, 0))
_nd_spec = pl.BlockSpec((R, D), lambda i: (i, 0))
_f_spec = pl.BlockSpec((R, 2 * D), lambda i: (i, 0))
_p0_spec = pl.BlockSpec((1, R, D), lambda i: (0, i, 0))
_p1_spec = pl.BlockSpec((1, R, D), lambda i: (1, i, 0))
_c0_spec = pl.BlockSpec((1, R, D), lambda i: (0, i, 0))
_c1_spec = pl.BlockSpec((1, R, D), lambda i: (1, i, 0))

_tc_out_shape = [
    jax.ShapeDtypeStruct((N, D), jnp.float32),
    jax.ShapeDtypeStruct((N, 2 * D), jnp.float32),
    jax.ShapeDtypeStruct((N, D), jnp.float32),
]
_tc_out_specs = [_nd_spec, _f_spec, _nd_spec]


def _tc_first(x, wcat, bf, bfs):
    return pl.pallas_call(
        _tc_first_body,
        grid=(N // R,),
        in_specs=[_nd_spec, _w_spec, _b_spec, _b_spec],
        out_specs=_tc_out_specs,
        out_shape=_tc_out_shape,
    )(x, wcat, bf, bfs)


def _tc_fuse(outp, p, cnt, wcat, bf, bfs):
    return pl.pallas_call(
        _tc_fuse_body,
        grid=(N // R,),
        in_specs=[_nd_spec, _p0_spec, _p1_spec, _c0_spec, _c1_spec,
                  _w_spec, _b_spec, _b_spec],
        out_specs=_tc_out_specs,
        out_shape=_tc_out_shape,
    )(outp, p, p, cnt, cnt, wcat, bf, bfs)


def _tc_final(outp, p, cnt):
    return pl.pallas_call(
        _tc_final_body,
        grid=(N // R,),
        in_specs=[_nd_spec, _p0_spec, _p1_spec, _c0_spec, _c1_spec],
        out_specs=_nd_spec,
        out_shape=jax.ShapeDtypeStruct((N, D), jnp.float32),
    )(outp, p, p, cnt, cnt)


def kernel(x, edge_index,
           W1, W1_skip, W1_film, b1_film, W1_film_skip, b1_film_skip,
           W2, W2_skip, W2_film, b2_film, W2_film_skip, b2_film_skip,
           W3, W3_skip, W3_film, b3_film, W3_film_skip, b3_film_skip,
           W4, W4_skip, W4_film, b4_film, W4_film_skip, b4_film_skip):
    src = edge_index[0]
    dst = edge_index[1]

    layers = [
        (W1, W1_skip, W1_film, b1_film, W1_film_skip, b1_film_skip),
        (W2, W2_skip, W2_film, b2_film, W2_film_skip, b2_film_skip),
        (W3, W3_skip, W3_film, b3_film, W3_film_skip, b3_film_skip),
        (W4, W4_skip, W4_film, b4_film, W4_film_skip, b4_film_skip),
    ]
    wcats = [jnp.concatenate([w, ws, wf, wfs], axis=1)
             for (w, ws, wf, _, wfs, _) in layers]
    bfs_ = [(bf.reshape(1, -1), bskip.reshape(1, -1))
            for (_, _, _, bf, _, bskip) in layers]

    (cnt,) = _cnt_kernel(dst)
    xw, f, outp = _tc_first(x, wcats[0], *bfs_[0])
    (p,) = _edge_kernel(src, dst, xw, f)
    for li in (1, 2, 3):
        xw, f, outp = _tc_fuse(outp, p, cnt, wcats[li], *bfs_[li])
        (p,) = _edge_kernel(src, dst, xw, f)
    return _tc_final(outp, p, cnt)


# speedup vs baseline: 1.2114x; 1.2114x over previous
import functools

import jax
import jax.numpy as jnp
from jax import lax
from jax.experimental import pallas as pl
from jax.experimental.pallas import tpu as pltpu
from jax.experimental.pallas import tpu_sc as plsc

N = 10000
E = 320000
D = 128

NC = 2
NS = 16
NW = NC * NS
EPW = E // NW
RPTA = 640
RPTB = N - 15 * RPTA


def _per_tile_rows(sid, fn):
    @pl.when(sid < NS - 1)
    def _():
        fn(sid * RPTA, RPTA)

    @pl.when(sid == NS - 1)
    def _():
        fn((NS - 1) * RPTA, RPTB)


_mesh = plsc.VectorSubcoreMesh(core_axis_name="c", subcore_axis_name="s")


K = 80
NCHUNK = EPW // K


def _fill_rows(buf, value):
    def body(j, _):
        for c in range(D // 16):
            buf[j, pl.ds(c * 16, 16)] = jnp.full((16,), value, jnp.float32)
        return 0
    lax.fori_loop(0, K, body, 0)


def _zero_table(sid, zbuf, table_sh):
    def zero_rows(r0, nr):
        for j in range(nr // K):
            pltpu.sync_copy(zbuf, table_sh.at[pl.ds(r0 + j * K, K)])
    _per_tile_rows(sid, zero_rows)


def _copy_out(sid, cid, table_sh, vbuf, out_h):
    def out_rows(r0, nr):
        for j in range(nr // K):
            pltpu.sync_copy(table_sh.at[pl.ds(r0 + j * K, K)], vbuf)
            pltpu.sync_copy(vbuf, out_h.at[cid, pl.ds(r0 + j * K, K)])
    _per_tile_rows(sid, out_rows)


NB = 5
CB = NCHUNK // NB


@functools.partial(
    pl.kernel,
    out_type=[jax.ShapeDtypeStruct((NC, N, D), jnp.float32)],
    mesh=_mesh,
    scratch_types=[
        pltpu.VMEM((CB, K), jnp.int32),
        pltpu.VMEM((CB, K), jnp.int32),
        pltpu.VMEM((K, D), jnp.float32),
        pltpu.VMEM((K, 2 * D), jnp.float32),
        pltpu.VMEM_SHARED((N, D), jnp.float32),
        pltpu.SemaphoreType.DMA,
        pltpu.SemaphoreType.DMA,
    ],
)
def _edge_kernel(src_h, dst_h, xw_h, f_h, p_h,
                 sidx, didx, xj, fv, agg_sh, sem1, sem2):
    cid = lax.axis_index("c")
    sid = lax.axis_index("s")
    wid = sid * NC + cid

    _fill_rows(xj, 0.0)
    _zero_table(sid, xj, agg_sh)
    plsc.subcore_barrier()

    def outer_body(ob, carry):
        pltpu.sync_copy(src_h.at[wid, ob], sidx)
        pltpu.sync_copy(dst_h.at[wid, ob], didx)

        def chunk_body(j, c2):
            cp1 = pltpu.async_copy(xw_h.at[sidx.at[j]], xj, sem1)
            cp2 = pltpu.async_copy(f_h.at[didx.at[j]], fv, sem2)
            cp1.wait()
            cp2.wait()

            @plsc.parallel_loop(0, K, 1, unroll=4)
            def _(e):
                for c in range(D // 16):
                    s = c * 16
                    xjv = xj[e, pl.ds(s, 16)]
                    b = fv[e, pl.ds(s, 16)]
                    g = fv[e, pl.ds(D + s, 16)]
                    xj[e, pl.ds(s, 16)] = jnp.maximum(g * xjv + b, 0.0)

            pltpu.sync_copy(xj, agg_sh.at[didx.at[j]], add=True)
            return c2

        lax.fori_loop(0, CB, chunk_body, 0)
        return carry

    lax.fori_loop(0, NB, outer_body, 0)
    plsc.subcore_barrier()
    _copy_out(sid, cid, agg_sh, xj, p_h)


@functools.partial(
    pl.kernel,
    out_type=[jax.ShapeDtypeStruct((NC, N, D), jnp.float32)],
    mesh=_mesh,
    scratch_types=[
        pltpu.VMEM((CB, K), jnp.int32),
        pltpu.VMEM((K, D), jnp.float32),
        pltpu.VMEM_SHARED((N, D), jnp.float32),
    ],
)
def _cnt_kernel(dst_h, cnt_h, didx, buf, cnt_sh):
    cid = lax.axis_index("c")
    sid = lax.axis_index("s")
    wid = sid * NC + cid

    _fill_rows(buf, 0.0)
    _zero_table(sid, buf, cnt_sh)
    _fill_rows(buf, 1.0)
    plsc.subcore_barrier()

    def outer_body(ob, carry):
        pltpu.sync_copy(dst_h.at[wid, ob], didx)

        def chunk_body(j, c2):
            pltpu.sync_copy(buf, cnt_sh.at[didx.at[j]], add=True)
            return c2

        lax.fori_loop(0, CB, chunk_body, 0)
        return carry

    lax.fori_loop(0, NB, outer_body, 0)
    plsc.subcore_barrier()
    _copy_out(sid, cid, cnt_sh, buf, cnt_h)


R = 1000


def _tc_common(h, wcat_ref, bf_ref, bfs_ref, xw_ref, f_ref, out_ref):
    r = jnp.dot(h, wcat_ref[...], preferred_element_type=jnp.float32)
    xw_ref[...] = r[:, 0:D]
    sk = r[:, D:2 * D]
    f_ref[...] = r[:, 2 * D:4 * D] + bf_ref[...]
    fs = r[:, 4 * D:6 * D] + bfs_ref[...]
    out_ref[...] = jnp.maximum(fs[:, D:2 * D] * sk + fs[:, 0:D], 0.0)


def _tc_first_body(x_ref, wcat_ref, bf_ref, bfs_ref, xw_ref, f_ref, out_ref):
    _tc_common(x_ref[...], wcat_ref, bf_ref, bfs_ref, xw_ref, f_ref, out_ref)


def _combine(outp_ref, p0_ref, p1_ref, c0_ref, c1_ref):
    cnt = c0_ref[0, :, 0:1] + c1_ref[0, :, 0:1]
    inv = 1.0 / jnp.maximum(cnt, 1.0)
    return outp_ref[...] + (p0_ref[0] + p1_ref[0]) * inv


def _tc_fuse_body(outp_ref, p0_ref, p1_ref, c0_ref, c1_ref,
                  wcat_ref, bf_ref, bfs_ref, xw_ref, f_ref, out_ref):
    h = jnp.maximum(_combine(outp_ref, p0_ref, p1_ref, c0_ref, c1_ref), 0.0)
    _tc_common(h, wcat_ref, bf_ref, bfs_ref, xw_ref, f_ref, out_ref)


def _tc_final_body(outp_ref, p0_ref, p1_ref, c0_ref, c1_ref, h_ref):
    h_ref[...] = _combine(outp_ref, p0_ref, p1_ref, c0_ref, c1_ref)


_w_spec = pl.BlockSpec((D, 6 * D), lambda i: (0, 0))
_b_spec = pl.BlockSpec((1, 2 * D), lambda i: (0, 0))
_nd_spec = pl.BlockSpec((R, D), lambda i: (i, 0))
_f_spec = pl.BlockSpec((R, 2 * D), lambda i: (i, 0))
_p0_spec = pl.BlockSpec((1, R, D), lambda i: (0, i, 0))
_p1_spec = pl.BlockSpec((1, R, D), lambda i: (1, i, 0))
_c0_spec = pl.BlockSpec((1, R, D), lambda i: (0, i, 0))
_c1_spec = pl.BlockSpec((1, R, D), lambda i: (1, i, 0))

_tc_out_shape = [
    jax.ShapeDtypeStruct((N, D), jnp.float32),
    jax.ShapeDtypeStruct((N, 2 * D), jnp.float32),
    jax.ShapeDtypeStruct((N, D), jnp.float32),
]
_tc_out_specs = [_nd_spec, _f_spec, _nd_spec]


def _tc_first(x, wcat, bf, bfs):
    return pl.pallas_call(
        _tc_first_body,
        grid=(N // R,),
        in_specs=[_nd_spec, _w_spec, _b_spec, _b_spec],
        out_specs=_tc_out_specs,
        out_shape=_tc_out_shape,
    )(x, wcat, bf, bfs)


def _tc_fuse(outp, p, cnt, wcat, bf, bfs):
    return pl.pallas_call(
        _tc_fuse_body,
        grid=(N // R,),
        in_specs=[_nd_spec, _p0_spec, _p1_spec, _c0_spec, _c1_spec,
                  _w_spec, _b_spec, _b_spec],
        out_specs=_tc_out_specs,
        out_shape=_tc_out_shape,
    )(outp, p, p, cnt, cnt, wcat, bf, bfs)


def _tc_final(outp, p, cnt):
    return pl.pallas_call(
        _tc_final_body,
        grid=(N // R,),
        in_specs=[_nd_spec, _p0_spec, _p1_spec, _c0_spec, _c1_spec],
        out_specs=_nd_spec,
        out_shape=jax.ShapeDtypeStruct((N, D), jnp.float32),
    )(outp, p, p, cnt, cnt)


def kernel(x, edge_index,
           W1, W1_skip, W1_film, b1_film, W1_film_skip, b1_film_skip,
           W2, W2_skip, W2_film, b2_film, W2_film_skip, b2_film_skip,
           W3, W3_skip, W3_film, b3_film, W3_film_skip, b3_film_skip,
           W4, W4_skip, W4_film, b4_film, W4_film_skip, b4_film_skip):
    src = edge_index[0].reshape(NW, NB, CB, K)
    dst = edge_index[1].reshape(NW, NB, CB, K)

    layers = [
        (W1, W1_skip, W1_film, b1_film, W1_film_skip, b1_film_skip),
        (W2, W2_skip, W2_film, b2_film, W2_film_skip, b2_film_skip),
        (W3, W3_skip, W3_film, b3_film, W3_film_skip, b3_film_skip),
        (W4, W4_skip, W4_film, b4_film, W4_film_skip, b4_film_skip),
    ]
    wcats = [jnp.concatenate([w, ws, wf, wfs], axis=1)
             for (w, ws, wf, _, wfs, _) in layers]
    bfs_ = [(bf.reshape(1, -1), bskip.reshape(1, -1))
            for (_, _, _, bf, _, bskip) in layers]

    (cnt,) = _cnt_kernel(dst)
    xw, f, outp = _tc_first(x, wcats[0], *bfs_[0])
    (p,) = _edge_kernel(src, dst, xw, f)
    for li in (1, 2, 3):
        xw, f, outp = _tc_fuse(outp, p, cnt, wcats[li], *bfs_[li])
        (p,) = _edge_kernel(src, dst, xw, f)
    return _tc_final(outp, p, cnt)


# speedup vs baseline: 1.7604x; 1.4532x over previous
import functools

import jax
import jax.numpy as jnp
from jax import lax
from jax.experimental import pallas as pl
from jax.experimental.pallas import tpu as pltpu
from jax.experimental.pallas import tpu_sc as plsc

N = 10000
E = 320000
D = 128

NC = 2
NS = 16
NW = NC * NS
EPW = E // NW
RPTA = 640
RPTB = N - 15 * RPTA


def _per_tile_rows(sid, fn):
    @pl.when(sid < NS - 1)
    def _():
        fn(sid * RPTA, RPTA)

    @pl.when(sid == NS - 1)
    def _():
        fn((NS - 1) * RPTA, RPTB)


_mesh = plsc.VectorSubcoreMesh(core_axis_name="c", subcore_axis_name="s")


K = 40
NCHUNK = EPW // K


def _fill_rows(buf, value):
    def body(j, _):
        for c in range(D // 16):
            buf[j, pl.ds(c * 16, 16)] = jnp.full((16,), value, jnp.float32)
        return 0
    lax.fori_loop(0, K, body, 0)


def _zero_table(sid, zbuf, table_sh):
    def zero_rows(r0, nr):
        for j in range(nr // K):
            pltpu.sync_copy(zbuf, table_sh.at[pl.ds(r0 + j * K, K)])
    _per_tile_rows(sid, zero_rows)


def _copy_out(sid, cid, table_sh, vbuf, out_h):
    def out_rows(r0, nr):
        for j in range(nr // K):
            pltpu.sync_copy(table_sh.at[pl.ds(r0 + j * K, K)], vbuf)
            pltpu.sync_copy(vbuf, out_h.at[cid, pl.ds(r0 + j * K, K)])
    _per_tile_rows(sid, out_rows)


NB = 5
CB = NCHUNK // NB


@functools.partial(
    pl.kernel,
    out_type=[jax.ShapeDtypeStruct((NC, N, D), jnp.float32)],
    mesh=_mesh,
    scratch_types=[
        pltpu.VMEM((CB, K), jnp.int32),
        pltpu.VMEM((CB, K), jnp.int32),
        pltpu.VMEM((K, D), jnp.float32),
        pltpu.VMEM((K, D), jnp.float32),
        pltpu.VMEM((K, 2 * D), jnp.float32),
        pltpu.VMEM((K, 2 * D), jnp.float32),
        pltpu.VMEM_SHARED((N, D), jnp.float32),
        pltpu.SemaphoreType.DMA,
        pltpu.SemaphoreType.DMA,
        pltpu.SemaphoreType.DMA,
        pltpu.SemaphoreType.DMA,
    ],
)
def _edge_kernel(src_h, dst_h, xw_h, f_h, p_h,
                 sidx, didx, xj0, xj1, fv0, fv1, agg_sh,
                 s1a, s2a, s1b, s2b):
    cid = lax.axis_index("c")
    sid = lax.axis_index("s")
    wid = sid * NC + cid

    _fill_rows(xj0, 0.0)
    _zero_table(sid, xj0, agg_sh)
    plsc.subcore_barrier()

    def start_gather(j, xjb, fvb, s1, s2):
        pltpu.async_copy(xw_h.at[sidx.at[j]], xjb, s1)
        pltpu.async_copy(f_h.at[didx.at[j]], fvb, s2)

    def process(j, xjb, fvb, s1, s2):
        pltpu.make_async_copy(xw_h.at[pl.ds(0, K)], xjb, s1).wait()
        pltpu.make_async_copy(f_h.at[pl.ds(0, K)], fvb, s2).wait()

        @plsc.parallel_loop(0, K, 1, unroll=4)
        def _(e):
            for c in range(D // 16):
                s = c * 16
                xjv = xjb[e, pl.ds(s, 16)]
                b = fvb[e, pl.ds(s, 16)]
                g = fvb[e, pl.ds(D + s, 16)]
                xjb[e, pl.ds(s, 16)] = jnp.maximum(g * xjv + b, 0.0)

        pltpu.sync_copy(xjb, agg_sh.at[didx.at[j]], add=True)

    def outer_body(ob, carry):
        pltpu.sync_copy(src_h.at[wid, ob], sidx)
        pltpu.sync_copy(dst_h.at[wid, ob], didx)
        start_gather(0, xj0, fv0, s1a, s2a)

        def pair_body(j2, c2):
            j0 = 2 * j2
            start_gather(j0 + 1, xj1, fv1, s1b, s2b)
            process(j0, xj0, fv0, s1a, s2a)

            @pl.when(j2 < CB // 2 - 1)
            def _():
                start_gather(j0 + 2, xj0, fv0, s1a, s2a)

            process(j0 + 1, xj1, fv1, s1b, s2b)
            return c2

        lax.fori_loop(0, CB // 2, pair_body, 0)
        return carry

    lax.fori_loop(0, NB, outer_body, 0)
    plsc.subcore_barrier()
    _copy_out(sid, cid, agg_sh, xj0, p_h)


@functools.partial(
    pl.kernel,
    out_type=[jax.ShapeDtypeStruct((NC, N, D), jnp.float32)],
    mesh=_mesh,
    scratch_types=[
        pltpu.VMEM((CB, K), jnp.int32),
        pltpu.VMEM((K, D), jnp.float32),
        pltpu.VMEM_SHARED((N, D), jnp.float32),
    ],
)
def _cnt_kernel(dst_h, cnt_h, didx, buf, cnt_sh):
    cid = lax.axis_index("c")
    sid = lax.axis_index("s")
    wid = sid * NC + cid

    _fill_rows(buf, 0.0)
    _zero_table(sid, buf, cnt_sh)
    _fill_rows(buf, 1.0)
    plsc.subcore_barrier()

    def outer_body(ob, carry):
        pltpu.sync_copy(dst_h.at[wid, ob], didx)

        def chunk_body(j, c2):
            pltpu.sync_copy(buf, cnt_sh.at[didx.at[j]], add=True)
            return c2

        lax.fori_loop(0, CB, chunk_body, 0)
        return carry

    lax.fori_loop(0, NB, outer_body, 0)
    plsc.subcore_barrier()
    _copy_out(sid, cid, cnt_sh, buf, cnt_h)


R = 1000


def _tc_common(h, wcat_ref, bf_ref, bfs_ref, xw_ref, f_ref, out_ref):
    r = jnp.dot(h, wcat_ref[...], preferred_element_type=jnp.float32)
    xw_ref[...] = r[:, 0:D]
    sk = r[:, D:2 * D]
    f_ref[...] = r[:, 2 * D:4 * D] + bf_ref[...]
    fs = r[:, 4 * D:6 * D] + bfs_ref[...]
    out_ref[...] = jnp.maximum(fs[:, D:2 * D] * sk + fs[:, 0:D], 0.0)


def _tc_first_body(x_ref, wcat_ref, bf_ref, bfs_ref, xw_ref, f_ref, out_ref):
    _tc_common(x_ref[...], wcat_ref, bf_ref, bfs_ref, xw_ref, f_ref, out_ref)


def _combine(outp_ref, p0_ref, p1_ref, c0_ref, c1_ref):
    cnt = c0_ref[0, :, 0:1] + c1_ref[0, :, 0:1]
    inv = 1.0 / jnp.maximum(cnt, 1.0)
    return outp_ref[...] + (p0_ref[0] + p1_ref[0]) * inv


def _tc_fuse_body(outp_ref, p0_ref, p1_ref, c0_ref, c1_ref,
                  wcat_ref, bf_ref, bfs_ref, xw_ref, f_ref, out_ref):
    h = jnp.maximum(_combine(outp_ref, p0_ref, p1_ref, c0_ref, c1_ref), 0.0)
    _tc_common(h, wcat_ref, bf_ref, bfs_ref, xw_ref, f_ref, out_ref)


def _tc_final_body(outp_ref, p0_ref, p1_ref, c0_ref, c1_ref, h_ref):
    h_ref[...] = _combine(outp_ref, p0_ref, p1_ref, c0_ref, c1_ref)


_w_spec = pl.BlockSpec((D, 6 * D), lambda i: (0, 0))
_b_spec = pl.BlockSpec((1, 2 * D), lambda i: (0, 0))
_nd_spec = pl.BlockSpec((R, D), lambda i: (i, 0))
_f_spec = pl.BlockSpec((R, 2 * D), lambda i: (i, 0))
_p0_spec = pl.BlockSpec((1, R, D), lambda i: (0, i, 0))
_p1_spec = pl.BlockSpec((1, R, D), lambda i: (1, i, 0))
_c0_spec = pl.BlockSpec((1, R, D), lambda i: (0, i, 0))
_c1_spec = pl.BlockSpec((1, R, D), lambda i: (1, i, 0))

_tc_out_shape = [
    jax.ShapeDtypeStruct((N, D), jnp.float32),
    jax.ShapeDtypeStruct((N, 2 * D), jnp.float32),
    jax.ShapeDtypeStruct((N, D), jnp.float32),
]
_tc_out_specs = [_nd_spec, _f_spec, _nd_spec]


def _tc_first(x, wcat, bf, bfs):
    return pl.pallas_call(
        _tc_first_body,
        grid=(N // R,),
        in_specs=[_nd_spec, _w_spec, _b_spec, _b_spec],
        out_specs=_tc_out_specs,
        out_shape=_tc_out_shape,
    )(x, wcat, bf, bfs)


def _tc_fuse(outp, p, cnt, wcat, bf, bfs):
    return pl.pallas_call(
        _tc_fuse_body,
        grid=(N // R,),
        in_specs=[_nd_spec, _p0_spec, _p1_spec, _c0_spec, _c1_spec,
                  _w_spec, _b_spec, _b_spec],
        out_specs=_tc_out_specs,
        out_shape=_tc_out_shape,
    )(outp, p, p, cnt, cnt, wcat, bf, bfs)


def _tc_final(outp, p, cnt):
    return pl.pallas_call(
        _tc_final_body,
        grid=(N // R,),
        in_specs=[_nd_spec, _p0_spec, _p1_spec, _c0_spec, _c1_spec],
        out_specs=_nd_spec,
        out_shape=jax.ShapeDtypeStruct((N, D), jnp.float32),
    )(outp, p, p, cnt, cnt)


def kernel(x, edge_index,
           W1, W1_skip, W1_film, b1_film, W1_film_skip, b1_film_skip,
           W2, W2_skip, W2_film, b2_film, W2_film_skip, b2_film_skip,
           W3, W3_skip, W3_film, b3_film, W3_film_skip, b3_film_skip,
           W4, W4_skip, W4_film, b4_film, W4_film_skip, b4_film_skip):
    src = edge_index[0].reshape(NW, NB, CB, K)
    dst = edge_index[1].reshape(NW, NB, CB, K)

    layers = [
        (W1, W1_skip, W1_film, b1_film, W1_film_skip, b1_film_skip),
        (W2, W2_skip, W2_film, b2_film, W2_film_skip, b2_film_skip),
        (W3, W3_skip, W3_film, b3_film, W3_film_skip, b3_film_skip),
        (W4, W4_skip, W4_film, b4_film, W4_film_skip, b4_film_skip),
    ]
    wcats = [jnp.concatenate([w, ws, wf, wfs], axis=1)
             for (w, ws, wf, _, wfs, _) in layers]
    bfs_ = [(bf.reshape(1, -1), bskip.reshape(1, -1))
            for (_, _, _, bf, _, bskip) in layers]

    (cnt,) = _cnt_kernel(dst)
    xw, f, outp = _tc_first(x, wcats[0], *bfs_[0])
    (p,) = _edge_kernel(src, dst, xw, f)
    for li in (1, 2, 3):
        xw, f, outp = _tc_fuse(outp, p, cnt, wcats[li], *bfs_[li])
        (p,) = _edge_kernel(src, dst, xw, f)
    return _tc_final(outp, p, cnt)
